# speedup vs baseline: 2.5606x; 2.5606x over previous
import functools

import jax
import jax.numpy as jnp
from jax.experimental import pallas as pl
from jax.experimental.pallas import tpu as pltpu

_B = 16384
_C = 1000
_BLK = 512


def _loss_body(x_ref, t_ref, out_ref):
    i = pl.program_id(0)
    x = x_ref[...]
    t = t_ref[...]
    col = jax.lax.broadcasted_iota(jnp.int32, x.shape, 1)
    e = jnp.exp(x)
    s = jnp.sum(e, axis=1, keepdims=True)
    xt = jnp.sum(jnp.where(col == t, x, 0.0), axis=1, keepdims=True)
    x0 = x[:, 0:1]
    e0 = jnp.exp(x0)
    neg = jnp.where(t != 0, e0, s - e0)
    li = jnp.log(neg + jnp.exp(xt)) - xt
    part = jnp.sum(li)

    @pl.when(i == 0)
    def _init():
        out_ref[0, 0] = 0.0

    out_ref[0, 0] += part


@functools.partial(jax.jit, static_argnames=())
def kernel(inputs, targets):
    t2 = targets.astype(jnp.int32).reshape(_B, 1)
    total = pl.pallas_call(
        _loss_body,
        grid=(_B // _BLK,),
        in_specs=[
            pl.BlockSpec((_BLK, _C), lambda i: (i, 0)),
            pl.BlockSpec((_BLK, 1), lambda i: (i, 0)),
        ],
        out_specs=pl.BlockSpec(
            (1, 1), lambda i: (0, 0), memory_space=pltpu.SMEM
        ),
        out_shape=jax.ShapeDtypeStruct((1, 1), jnp.float32),
    )(inputs, t2)
    return (total[0, 0] / _B).astype(inputs.dtype)
